# baseline (speedup 1.0000x reference)
import functools

import jax
import jax.numpy as jnp
from jax import lax
from jax.experimental import pallas as pl
from jax.experimental.pallas import tpu as pltpu
from jax.experimental.pallas import tpu_sc as plsc

_D = 2048
_E = 16
_T = 16384
_BLK = 1024

_NC = 2
_NS = 16
_L = 16
_NW = _NC * _NS
_TPW = _T // _NW
_G = _TPW // _L

_NEG = -3e38


_DH = _D // 2


def _logits_body(xa_ref, xb_ref, wta_ref, wtb_ref, b_ref, out_ref):
    acc = jnp.dot(xa_ref[...], wta_ref[...],
                  preferred_element_type=jnp.float32)
    acc = acc + jnp.dot(xb_ref[...], wtb_ref[...],
                        preferred_element_type=jnp.float32)
    out_ref[...] = acc + b_ref[...]


def _compute_logits(x_flat, wt, b2d):
    xa = x_flat[:, :_DH]
    xb = x_flat[:, _DH:]
    return pl.pallas_call(
        _logits_body,
        grid=(_T // _BLK,),
        in_specs=[
            pl.BlockSpec((_BLK, _DH), lambda i: (i, 0)),
            pl.BlockSpec((_BLK, _DH), lambda i: (i, 0)),
            pl.BlockSpec((_DH, _E), lambda i: (0, 0)),
            pl.BlockSpec((_DH, _E), lambda i: (0, 0)),
            pl.BlockSpec((1, _E), lambda i: (0, 0)),
        ],
        out_specs=pl.BlockSpec((_BLK, _E), lambda i: (i, 0)),
        out_shape=jax.ShapeDtypeStruct((_T, _E), jnp.float32),
        compiler_params=pltpu.CompilerParams(
            dimension_semantics=("parallel",)),
    )(xa, xb, wt[:_DH], wt[_DH:], b2d)


def _routing_body(logits_hbm, disp_hbm, buf_in, buf_out):
    wid = lax.axis_index("s") * _NC + lax.axis_index("c")
    base = wid * _TPW * _E
    pltpu.sync_copy(logits_hbm.at[pl.ds(base, _TPW * _E)], buf_in)

    lane16 = lax.broadcasted_iota(jnp.int32, (_L,), 0) * _E

    def group(g, carry):
        flat0 = g * (_L * _E) + lane16
        ls = [plsc.load_gather(buf_in, [flat0 + e]) for e in range(_E)]
        m1 = ls[0]
        for e in range(1, _E):
            m1 = jnp.maximum(m1, ls[e])
        i1 = jnp.full((_L,), _E, jnp.int32)
        for e in range(_E - 1, -1, -1):
            i1 = jnp.where(ls[e] == m1, e, i1)
        m2 = jnp.full((_L,), _NEG, jnp.float32)
        for e in range(_E):
            m2 = jnp.maximum(m2, jnp.where(i1 == e, _NEG, ls[e]))
        i2 = jnp.full((_L,), _E, jnp.int32)
        for e in range(_E - 1, -1, -1):
            i2 = jnp.where((ls[e] == m2) & (i1 != e), e, i2)
        s = jnp.exp(m2 - m1)
        denom = 1.0 + s
        v1 = 1.0 / denom
        v2 = s / denom
        zero = jnp.zeros((_L,), jnp.float32)
        for e in range(_E):
            val = jnp.where(i1 == e, v1, jnp.where(i2 == e, v2, zero))
            plsc.store_scatter(buf_out, [flat0 + e], val)
        return carry

    lax.fori_loop(0, _G, group, 0)
    pltpu.sync_copy(buf_out, disp_hbm.at[pl.ds(base, _TPW * _E)])


_routing = pl.kernel(
    _routing_body,
    out_type=jax.ShapeDtypeStruct((_T * _E,), jnp.float32),
    mesh=plsc.VectorSubcoreMesh(core_axis_name="c", subcore_axis_name="s"),
    compiler_params=pltpu.CompilerParams(needs_layout_passes=False),
    scratch_types=[
        pltpu.VMEM((_TPW * _E,), jnp.float32),
        pltpu.VMEM((_TPW * _E,), jnp.float32),
    ],
)


def kernel(x, W, b):
    x_flat = x.reshape(-1, _D)
    logits = _compute_logits(x_flat, W.T, b.reshape(1, _E))
    dispatch = _routing(logits.reshape(-1)).reshape(_T, _E)
    mask = dispatch > 0
    return (logits, dispatch, mask)


# speedup vs baseline: 1.6901x; 1.6901x over previous
import functools

import jax
import jax.numpy as jnp
from jax import lax
from jax.experimental import pallas as pl
from jax.experimental.pallas import tpu as pltpu
from jax.experimental.pallas import tpu_sc as plsc

_D = 2048
_E = 16
_T = 16384
_BLK = 1024

_NC = 2
_NS = 16
_L = 16
_NW = _NC * _NS
_TPW = _T // _NW
_G = _TPW // _L

_NEG = -3e38


_HALF_BLKS = _T // _BLK // 2


def _logits_body(xa_ref, xb_ref, wt_ref, b_ref, outa_ref, outb_ref):
    outa_ref[...] = jnp.dot(xa_ref[...], wt_ref[...],
                            preferred_element_type=jnp.float32) + b_ref[...]
    outb_ref[...] = jnp.dot(xb_ref[...], wt_ref[...],
                            preferred_element_type=jnp.float32) + b_ref[...]


def _compute_logits(x_flat, wt, b2d):
    outa, outb = pl.pallas_call(
        _logits_body,
        grid=(_HALF_BLKS,),
        in_specs=[
            pl.BlockSpec((_BLK, _D), lambda i: (i, 0)),
            pl.BlockSpec((_BLK, _D), lambda i: (i + _HALF_BLKS, 0)),
            pl.BlockSpec((_D, _E), lambda i: (0, 0)),
            pl.BlockSpec((1, _E), lambda i: (0, 0)),
        ],
        out_specs=[
            pl.BlockSpec((_BLK, _E), lambda i: (i, 0)),
            pl.BlockSpec((_BLK, _E), lambda i: (i, 0)),
        ],
        out_shape=[
            jax.ShapeDtypeStruct((_T // 2, _E), jnp.float32),
            jax.ShapeDtypeStruct((_T // 2, _E), jnp.float32),
        ],
        compiler_params=pltpu.CompilerParams(
            dimension_semantics=("parallel",)),
    )(x_flat, x_flat, wt, b2d)
    return jnp.concatenate([outa, outb], axis=0)


def _routing_body(logits_hbm, disp_hbm, buf_in, buf_out):
    wid = lax.axis_index("s") * _NC + lax.axis_index("c")
    base = wid * _TPW * _E
    pltpu.sync_copy(logits_hbm.at[pl.ds(base, _TPW * _E)], buf_in)

    lane16 = lax.broadcasted_iota(jnp.int32, (_L,), 0) * _E

    def group(g, carry):
        flat0 = g * (_L * _E) + lane16
        ls = [plsc.load_gather(buf_in, [flat0 + e]) for e in range(_E)]
        m1 = ls[0]
        for e in range(1, _E):
            m1 = jnp.maximum(m1, ls[e])
        i1 = jnp.full((_L,), _E, jnp.int32)
        for e in range(_E - 1, -1, -1):
            i1 = jnp.where(ls[e] == m1, e, i1)
        m2 = jnp.full((_L,), _NEG, jnp.float32)
        for e in range(_E):
            m2 = jnp.maximum(m2, jnp.where(i1 == e, _NEG, ls[e]))
        i2 = jnp.full((_L,), _E, jnp.int32)
        for e in range(_E - 1, -1, -1):
            i2 = jnp.where((ls[e] == m2) & (i1 != e), e, i2)
        s = jnp.exp(m2 - m1)
        denom = 1.0 + s
        v1 = 1.0 / denom
        v2 = s / denom
        zero = jnp.zeros((_L,), jnp.float32)
        for e in range(_E):
            val = jnp.where(i1 == e, v1, jnp.where(i2 == e, v2, zero))
            plsc.store_scatter(buf_out, [flat0 + e], val)
        return carry

    lax.fori_loop(0, _G, group, 0)
    pltpu.sync_copy(buf_out, disp_hbm.at[pl.ds(base, _TPW * _E)])


_routing = pl.kernel(
    _routing_body,
    out_type=jax.ShapeDtypeStruct((_T * _E,), jnp.float32),
    mesh=plsc.VectorSubcoreMesh(core_axis_name="c", subcore_axis_name="s"),
    compiler_params=pltpu.CompilerParams(needs_layout_passes=False),
    scratch_types=[
        pltpu.VMEM((_TPW * _E,), jnp.float32),
        pltpu.VMEM((_TPW * _E,), jnp.float32),
    ],
)


def kernel(x, W, b):
    x_flat = x.reshape(-1, _D)
    logits = _compute_logits(x_flat, W.T, b.reshape(1, _E))
    dispatch = _routing(logits.reshape(-1)).reshape(_T, _E)
    mask = dispatch > 0
    return (logits, dispatch, mask)


# speedup vs baseline: 2.0049x; 1.1863x over previous
import functools

import jax
import jax.numpy as jnp
from jax import lax
from jax.experimental import pallas as pl
from jax.experimental.pallas import tpu as pltpu
from jax.experimental.pallas import tpu_sc as plsc

_D = 2048
_E = 16
_T = 16384
_BLK = 1024

_NC = 2
_NS = 16
_L = 16
_NW = _NC * _NS
_TPW = _T // _NW
_G = _TPW // _L

_NEG = -3e38


def _logits_body(x_ref, wt_ref, b_ref, out_ref):
    out_ref[...] = jnp.dot(x_ref[...], wt_ref[...],
                           preferred_element_type=jnp.float32) + b_ref[...]


def _compute_logits(x_flat, wt, b2d, n_tok, tok_off):
    blk_off = tok_off // _BLK
    return pl.pallas_call(
        _logits_body,
        grid=(n_tok // _BLK,),
        in_specs=[
            pl.BlockSpec((_BLK, _D), lambda i: (i + blk_off, 0)),
            pl.BlockSpec((_D, _E), lambda i: (0, 0)),
            pl.BlockSpec((1, _E), lambda i: (0, 0)),
        ],
        out_specs=pl.BlockSpec((_BLK, _E), lambda i: (i, 0)),
        out_shape=jax.ShapeDtypeStruct((n_tok, _E), jnp.float32),
        compiler_params=pltpu.CompilerParams(
            dimension_semantics=("parallel",)),
    )(x_flat, wt, b2d)


@functools.lru_cache(maxsize=None)
def _make_routing(n_tok):
    tpw = n_tok // _NW
    ng = tpw // _L

    def body(logits_hbm, disp_hbm, buf_in, buf_out):
        wid = lax.axis_index("s") * _NC + lax.axis_index("c")
        base = wid * tpw * _E
        pltpu.sync_copy(logits_hbm.at[pl.ds(base, tpw * _E)], buf_in)

        lane16 = lax.broadcasted_iota(jnp.int32, (_L,), 0) * _E

        def group(g, carry):
            flat0 = g * (_L * _E) + lane16
            ls = [plsc.load_gather(buf_in, [flat0 + e]) for e in range(_E)]
            m1 = ls[0]
            for e in range(1, _E):
                m1 = jnp.maximum(m1, ls[e])
            i1 = jnp.full((_L,), _E, jnp.int32)
            for e in range(_E - 1, -1, -1):
                i1 = jnp.where(ls[e] == m1, e, i1)
            m2 = jnp.full((_L,), _NEG, jnp.float32)
            for e in range(_E):
                m2 = jnp.maximum(m2, jnp.where(i1 == e, _NEG, ls[e]))
            i2 = jnp.full((_L,), _E, jnp.int32)
            for e in range(_E - 1, -1, -1):
                i2 = jnp.where((ls[e] == m2) & (i1 != e), e, i2)
            s = jnp.exp(m2 - m1)
            denom = 1.0 + s
            v1 = 1.0 / denom
            v2 = s / denom
            zero = jnp.zeros((_L,), jnp.float32)
            for e in range(_E):
                val = jnp.where(i1 == e, v1, jnp.where(i2 == e, v2, zero))
                plsc.store_scatter(buf_out, [flat0 + e], val)
            return carry

        lax.fori_loop(0, ng, group, 0)
        pltpu.sync_copy(buf_out, disp_hbm.at[pl.ds(base, tpw * _E)])

    return pl.kernel(
        body,
        out_type=jax.ShapeDtypeStruct((n_tok * _E,), jnp.float32),
        mesh=plsc.VectorSubcoreMesh(core_axis_name="c", subcore_axis_name="s"),
        compiler_params=pltpu.CompilerParams(needs_layout_passes=False),
        scratch_types=[
            pltpu.VMEM((tpw * _E,), jnp.float32),
            pltpu.VMEM((tpw * _E,), jnp.float32),
        ],
    )


_N_CHUNK = 2


def kernel(x, W, b):
    x_flat = x.reshape(-1, _D)
    wt = W.T
    b2d = b.reshape(1, _E)
    ctok = _T // _N_CHUNK
    route = _make_routing(ctok)
    logits_parts = []
    disp_parts = []
    for c in range(_N_CHUNK):
        lg = _compute_logits(x_flat, wt, b2d, ctok, c * ctok)
        logits_parts.append(lg)
        disp_parts.append(route(lg.reshape(-1)).reshape(ctok, _E))
    logits = jnp.concatenate(logits_parts, axis=0)
    dispatch = jnp.concatenate(disp_parts, axis=0)
    mask = dispatch > 0
    return (logits, dispatch, mask)
